# baseline (speedup 1.0000x reference)
import functools

import jax
import jax.numpy as jnp
from jax import lax
from jax.experimental import pallas as pl
from jax.experimental.pallas import tpu as pltpu, tpu_sc as plsc

HIDDEN = 128
NC, NS, LANES = 2, 16, 16
NW = NC * NS
CH = 128


def _comb_body(pos_ref, turn_ref, out_ref):
    p = pos_ref[...]
    t = turn_ref[...]
    out_ref[...] = p[:, None, :] + t[None, :, :]


def _body(tok_idx, turn_idx, pos_idx, tok_tab, comb_tab, out,
          tokidx, turnidx, posidx, cidx, buf_t, buf_c, buf_o,
          gsem0, gsem1, wsem0, wsem1,
          *, rows_per_worker, n_turn):
    wid = lax.axis_index("s") * NC + lax.axis_index("c")
    wbase = wid * rows_per_worker
    n_chunks = rows_per_worker // CH
    n_pairs = n_chunks // 2
    gsem = (gsem0, gsem1)
    wsem = (wsem0, wsem1)

    def issue(g, b):
        base = wbase + g * CH
        pltpu.sync_copy(tok_idx.at[pl.ds(base, CH)], tokidx.at[b])
        pltpu.sync_copy(turn_idx.at[pl.ds(base, CH)], turnidx.at[b])
        pltpu.sync_copy(pos_idx.at[pl.ds(base, CH)], posidx.at[b])
        for i in range(CH // LANES):
            s = pl.ds(i * LANES, LANES)
            cidx[b, s] = posidx[b, s] * n_turn + turnidx[b, s]
        pltpu.async_copy(tok_tab.at[tokidx.at[b]], buf_t.at[b], gsem[b])
        pltpu.async_copy(comb_tab.at[cidx.at[b]], buf_c.at[b], gsem[b])

    def wait_gathers(b):
        pltpu.make_async_copy(tok_tab.at[pl.ds(0, CH)], buf_t.at[b],
                              gsem[b]).wait()
        pltpu.make_async_copy(tok_tab.at[pl.ds(0, CH)], buf_c.at[b],
                              gsem[b]).wait()

    def add_and_store(g, b):
        @plsc.parallel_loop(0, CH, step=1, unroll=2)
        def row_body(r):
            for c in range(HIDDEN // LANES):
                s = pl.ds(c * LANES, LANES)
                buf_o[b, r, s] = buf_t[b, r, s] + buf_c[b, r, s]

        pltpu.async_copy(buf_o.at[b], out.at[pl.ds(wbase + g * CH, CH)],
                         wsem[b])

    def wait_wb(b):
        pltpu.make_async_copy(buf_o.at[b], out.at[pl.ds(wbase, CH)],
                              wsem[b]).wait()

    issue(0, 0)

    def pair_body(p, _):
        g0 = 2 * p

        issue(g0 + 1, 1)
        wait_gathers(0)

        @pl.when(p > 0)
        def _():
            wait_wb(0)

        add_and_store(g0, 0)

        @pl.when(p < n_pairs - 1)
        def _():
            issue(g0 + 2, 0)

        wait_gathers(1)

        @pl.when(p > 0)
        def _():
            wait_wb(1)

        add_and_store(g0 + 1, 1)
        return 0

    lax.fori_loop(0, n_pairs, pair_body, 0, unroll=False)
    wait_wb(0)
    wait_wb(1)


def kernel(token_inp, turn_inp, pos_inp, tok_table, pos_table, turn_table):
    B, L = token_inp.shape
    N = B * L
    assert N % (NW * CH * 2) == 0
    rows_per_worker = N // NW
    P = pos_table.shape[0]
    T = turn_table.shape[0]

    comb = pl.pallas_call(
        _comb_body,
        out_shape=jax.ShapeDtypeStruct((P, T, HIDDEN), jnp.float32),
    )(pos_table, turn_table).reshape(P * T, HIDDEN)

    mesh = plsc.VectorSubcoreMesh(core_axis_name="c", subcore_axis_name="s",
                                  num_cores=NC, num_subcores=NS)
    k = pl.kernel(
        functools.partial(_body, rows_per_worker=rows_per_worker, n_turn=T),
        out_type=jax.ShapeDtypeStruct((N, HIDDEN), jnp.float32),
        mesh=mesh,
        scratch_types=[
            pltpu.VMEM((2, CH), jnp.int32),
            pltpu.VMEM((2, CH), jnp.int32),
            pltpu.VMEM((2, CH), jnp.int32),
            pltpu.VMEM((2, CH), jnp.int32),
            pltpu.VMEM((2, CH, HIDDEN), jnp.float32),
            pltpu.VMEM((2, CH, HIDDEN), jnp.float32),
            pltpu.VMEM((2, CH, HIDDEN), jnp.float32),
            pltpu.SemaphoreType.DMA,
            pltpu.SemaphoreType.DMA,
            pltpu.SemaphoreType.DMA,
            pltpu.SemaphoreType.DMA,
        ],
    )
    out = k(token_inp.reshape(N), turn_inp.reshape(N), pos_inp.reshape(N),
            tok_table, comb)
    return out.reshape(B, L, HIDDEN)


# speedup vs baseline: 1.2300x; 1.2300x over previous
import functools

import jax
import jax.numpy as jnp
from jax import lax
from jax.experimental import pallas as pl
from jax.experimental.pallas import tpu as pltpu, tpu_sc as plsc

HIDDEN = 128
NC, NS, LANES = 2, 16, 16
NW = NC * NS
CH = 128


def _comb_body(pos_ref, turn_ref, out_ref):
    p = pos_ref[...]
    t = turn_ref[...]
    out_ref[...] = p[:, None, :] + t[None, :, :]


def _body(idx_all, tok_tab, comb_tab, out,
          idx3, cidx, buf_t, buf_c, buf_o,
          gsem0, gsem1, wsem0, wsem1,
          *, rows_per_worker, n_turn):
    wid = lax.axis_index("s") * NC + lax.axis_index("c")
    wbase = wid * rows_per_worker
    n_chunks = rows_per_worker // CH
    n_pairs = n_chunks // 2
    gsem = (gsem0, gsem1)
    wsem = (wsem0, wsem1)

    def issue(g, b):
        slab = (wid * n_chunks + g) * (3 * CH)
        pltpu.sync_copy(idx_all.at[pl.ds(slab, 3 * CH)], idx3.at[b])
        for i in range(CH // LANES):
            s = pl.ds(i * LANES, LANES)
            cidx[b, s] = (idx3[b, pl.ds(2 * CH + i * LANES, LANES)] * n_turn
                          + idx3[b, pl.ds(CH + i * LANES, LANES)])
        pltpu.async_copy(tok_tab.at[idx3.at[b, pl.ds(0, CH)]], buf_t.at[b],
                         gsem[b])
        pltpu.async_copy(comb_tab.at[cidx.at[b]], buf_c.at[b], gsem[b])

    def wait_gathers(b):
        pltpu.make_async_copy(tok_tab.at[pl.ds(0, CH)], buf_t.at[b],
                              gsem[b]).wait()
        pltpu.make_async_copy(tok_tab.at[pl.ds(0, CH)], buf_c.at[b],
                              gsem[b]).wait()

    def add_and_store(g, b):
        @plsc.parallel_loop(0, CH, step=1)
        def row_body(r):
            for c in range(HIDDEN // LANES):
                s = pl.ds(c * LANES, LANES)
                buf_o[b, r, s] = buf_t[b, r, s] + buf_c[b, r, s]

        pltpu.async_copy(buf_o.at[b], out.at[pl.ds(wbase + g * CH, CH)],
                         wsem[b])

    def wait_wb(b):
        pltpu.make_async_copy(buf_o.at[b], out.at[pl.ds(wbase, CH)],
                              wsem[b]).wait()

    issue(0, 0)

    def pair_body(p, _):
        g0 = 2 * p

        issue(g0 + 1, 1)
        wait_gathers(0)

        @pl.when(p > 0)
        def _():
            wait_wb(0)

        add_and_store(g0, 0)

        @pl.when(p < n_pairs - 1)
        def _():
            issue(g0 + 2, 0)

        wait_gathers(1)

        @pl.when(p > 0)
        def _():
            wait_wb(1)

        add_and_store(g0 + 1, 1)
        return 0

    lax.fori_loop(0, n_pairs, pair_body, 0, unroll=False)
    wait_wb(0)
    wait_wb(1)


def kernel(token_inp, turn_inp, pos_inp, tok_table, pos_table, turn_table):
    B, L = token_inp.shape
    N = B * L
    assert N % (NW * CH * 2) == 0
    rows_per_worker = N // NW
    P = pos_table.shape[0]
    T = turn_table.shape[0]

    comb = pl.pallas_call(
        _comb_body,
        out_shape=jax.ShapeDtypeStruct((P, T, HIDDEN), jnp.float32),
    )(pos_table, turn_table).reshape(P * T, HIDDEN)

    mesh = plsc.VectorSubcoreMesh(core_axis_name="c", subcore_axis_name="s",
                                  num_cores=NC, num_subcores=NS)
    k = pl.kernel(
        functools.partial(_body, rows_per_worker=rows_per_worker, n_turn=T),
        out_type=jax.ShapeDtypeStruct((N, HIDDEN), jnp.float32),
        mesh=mesh,
        scratch_types=[
            pltpu.VMEM((2, 3 * CH), jnp.int32),
            pltpu.VMEM((2, CH), jnp.int32),
            pltpu.VMEM((2, CH, HIDDEN), jnp.float32),
            pltpu.VMEM((2, CH, HIDDEN), jnp.float32),
            pltpu.VMEM((2, CH, HIDDEN), jnp.float32),
            pltpu.SemaphoreType.DMA,
            pltpu.SemaphoreType.DMA,
            pltpu.SemaphoreType.DMA,
            pltpu.SemaphoreType.DMA,
        ],
    )
    n_chunks = rows_per_worker // CH
    idx_all = jnp.stack([token_inp.reshape(N), turn_inp.reshape(N),
                         pos_inp.reshape(N)])
    idx_all = idx_all.reshape(3, NW, n_chunks, CH).transpose(1, 2, 0, 3)
    out = k(idx_all.reshape(3 * N), tok_table, comb)
    return out.reshape(B, L, HIDDEN)


# speedup vs baseline: 1.2580x; 1.0228x over previous
import functools

import jax
import jax.numpy as jnp
from jax import lax
from jax.experimental import pallas as pl
from jax.experimental.pallas import tpu as pltpu, tpu_sc as plsc

HIDDEN = 128
NC, NS, LANES = 2, 16, 16
NW = NC * NS
CH = 128


def _comb_body(pos_ref, turn_ref, out_ref):
    p = pos_ref[...]
    t = turn_ref[...]
    out_ref[...] = p[:, None, :] + t[None, :, :]


def _body(idx_all, tok_tab, comb_tab, out,
          idx3, cidx, buf_t, buf_c, buf_o,
          gsem0, gsem1, wsem0, wsem1, isem0, isem1,
          *, rows_per_worker, n_turn):
    wid = lax.axis_index("s") * NC + lax.axis_index("c")
    wbase = wid * rows_per_worker
    n_chunks = rows_per_worker // CH
    n_pairs = n_chunks // 2
    gsem = (gsem0, gsem1)
    wsem = (wsem0, wsem1)
    isem = (isem0, isem1)

    def idx_fetch(g, b):
        slab = (wid * n_chunks + g) * (3 * CH)
        pltpu.async_copy(idx_all.at[pl.ds(slab, 3 * CH)], idx3.at[b], isem[b])

    def wait_idx(b):
        pltpu.make_async_copy(idx_all.at[pl.ds(0, 3 * CH)], idx3.at[b],
                              isem[b]).wait()

    def issue(g, b):
        for i in range(CH // LANES):
            s = pl.ds(i * LANES, LANES)
            cidx[b, s] = (idx3[b, pl.ds(2 * CH + i * LANES, LANES)] * n_turn
                          + idx3[b, pl.ds(CH + i * LANES, LANES)])
        pltpu.async_copy(tok_tab.at[idx3.at[b, pl.ds(0, CH)]], buf_t.at[b],
                         gsem[b])
        pltpu.async_copy(comb_tab.at[cidx.at[b]], buf_c.at[b], gsem[b])

    def wait_gathers(b):
        pltpu.make_async_copy(tok_tab.at[pl.ds(0, CH)], buf_t.at[b],
                              gsem[b]).wait()
        pltpu.make_async_copy(tok_tab.at[pl.ds(0, CH)], buf_c.at[b],
                              gsem[b]).wait()

    def add_and_store(g, b):
        @plsc.parallel_loop(0, CH, step=1)
        def row_body(r):
            for c in range(HIDDEN // LANES):
                s = pl.ds(c * LANES, LANES)
                buf_o[b, r, s] = buf_t[b, r, s] + buf_c[b, r, s]

        pltpu.async_copy(buf_o.at[b], out.at[pl.ds(wbase + g * CH, CH)],
                         wsem[b])

    def wait_wb(b):
        pltpu.make_async_copy(buf_o.at[b], out.at[pl.ds(wbase, CH)],
                              wsem[b]).wait()

    idx_fetch(0, 0)
    idx_fetch(1, 1)
    wait_idx(0)
    issue(0, 0)

    def pair_body(p, _):
        g0 = 2 * p

        wait_idx(1)
        issue(g0 + 1, 1)
        wait_gathers(0)

        @pl.when(p < n_pairs - 1)
        def _():
            idx_fetch(g0 + 2, 0)

        @pl.when(p > 0)
        def _():
            wait_wb(0)

        add_and_store(g0, 0)

        @pl.when(p < n_pairs - 1)
        def _():
            wait_idx(0)
            issue(g0 + 2, 0)

        wait_gathers(1)

        @pl.when(p < n_pairs - 1)
        def _():
            idx_fetch(g0 + 3, 1)

        @pl.when(p > 0)
        def _():
            wait_wb(1)

        add_and_store(g0 + 1, 1)
        return 0

    lax.fori_loop(0, n_pairs, pair_body, 0, unroll=False)
    wait_wb(0)
    wait_wb(1)


def kernel(token_inp, turn_inp, pos_inp, tok_table, pos_table, turn_table):
    B, L = token_inp.shape
    N = B * L
    assert N % (NW * CH * 2) == 0
    rows_per_worker = N // NW
    P = pos_table.shape[0]
    T = turn_table.shape[0]

    comb = pl.pallas_call(
        _comb_body,
        out_shape=jax.ShapeDtypeStruct((P, T, HIDDEN), jnp.float32),
    )(pos_table, turn_table).reshape(P * T, HIDDEN)

    mesh = plsc.VectorSubcoreMesh(core_axis_name="c", subcore_axis_name="s",
                                  num_cores=NC, num_subcores=NS)
    k = pl.kernel(
        functools.partial(_body, rows_per_worker=rows_per_worker, n_turn=T),
        out_type=jax.ShapeDtypeStruct((N, HIDDEN), jnp.float32),
        mesh=mesh,
        scratch_types=[
            pltpu.VMEM((2, 3 * CH), jnp.int32),
            pltpu.VMEM((2, CH), jnp.int32),
            pltpu.VMEM((2, CH, HIDDEN), jnp.float32),
            pltpu.VMEM((2, CH, HIDDEN), jnp.float32),
            pltpu.VMEM((2, CH, HIDDEN), jnp.float32),
            pltpu.SemaphoreType.DMA,
            pltpu.SemaphoreType.DMA,
            pltpu.SemaphoreType.DMA,
            pltpu.SemaphoreType.DMA,
            pltpu.SemaphoreType.DMA,
            pltpu.SemaphoreType.DMA,
        ],
    )
    n_chunks = rows_per_worker // CH
    idx_all = jnp.stack([token_inp.reshape(N), turn_inp.reshape(N),
                         pos_inp.reshape(N)])
    idx_all = idx_all.reshape(3, NW, n_chunks, CH).transpose(1, 2, 0, 3)
    out = k(idx_all.reshape(3 * N), tok_table, comb)
    return out.reshape(B, L, HIDDEN)
